# Initial kernel scaffold; baseline (speedup 1.0000x reference)
import jax
import jax.numpy as jnp
from jax.experimental import pallas as pl


def kernel(x, edge_index, edge_weight, enc_w, enc_b, node_emb, wq, bq, wk, bk, wv, bv, wo, bo, diff_w, diff_b, dec_w, dec_b):
    raise NotImplementedError("write your pallas kernel here")


# speedup vs baseline: 1.0526x; 1.0526x over previous
import functools
import jax
import jax.numpy as jnp
from jax import lax
from jax.experimental import pallas as pl

B, T, N, F_IN = 2, 12, 50000, 1
E = 1600000
H = 32
HEADS = 8
HD = H // HEADS
HORIZON = 12
K = 2

BN = 1024
NPAD = ((N + BN - 1) // BN) * BN


def _attn_block(x_ref, nembT_ref, enc_w_ref, enc_b_ref, wq_ref, bq_ref,
                wk_ref, wv_ref, bv_ref, wo_ref, bo_ref, out_ref):
    f32 = jnp.float32
    dg = lax.dot_general
    embT = nembT_ref[...] + enc_b_ref[...]
    wq = wq_ref[...]
    wv = wv_ref[...]
    wo = wo_ref[...]
    enc_w = enc_w_ref[...]
    qvecC = dg(wq, enc_w, (((0,), (1,)), ((), ())), preferred_element_type=f32)
    kvecC = dg(wk_ref[...], enc_w, (((0,), (1,)), ((), ())), preferred_element_type=f32)
    vvec_row = dg(enc_w, wv, (((1,), (0,)), ((), ())), preferred_element_type=f32)
    qbaseT = dg(wq, embT, (((0,), (0,)), ((), ())), preferred_element_type=f32) + bq_ref[...]
    vbaseT = dg(wv, embT, (((0,), (0,)), ((), ())), preferred_element_type=f32) + bv_ref[...]
    hid = lax.broadcasted_iota(jnp.int32, (HEADS, H), 0)
    cid = lax.broadcasted_iota(jnp.int32, (HEADS, H), 1)
    Ssel = jnp.where(cid // HD == hid, 1.0, 0.0).astype(f32)
    alpha8 = dg(Ssel, qvecC * kvecC, (((1,), (0,)), ((), ())), preferred_element_type=f32)
    betaT = dg(Ssel, qbaseT * kvecC, (((1,), (0,)), ((), ())), preferred_element_type=f32)
    wov = dg(Ssel * vvec_row, wo, (((1,), (0,)), ((), ())), preferred_element_type=f32)
    obase = dg(vbaseT, wo, (((0,), (0,)), ((), ())), preferred_element_type=f32) + bo_ref[...]
    for b in range(B):
        xbt = x_ref[b * T:(b + 1) * T, :]
        xlast = x_ref[b * T + T - 1:b * T + T, :]
        aT = alpha8 * xlast + betaT
        srows = []
        for h in range(HEADS):
            lg = aT[h:h + 1, :] * xbt * 0.5
            m = jnp.max(lg, axis=0, keepdims=True)
            e = jnp.exp(lg - m)
            den = jnp.sum(e, axis=0, keepdims=True)
            srows.append(jnp.sum(e * xbt, axis=0, keepdims=True) / den)
        s8 = jnp.concatenate(srows, axis=0)
        hl = dg(s8, wov, (((0,), (0,)), ((), ())), preferred_element_type=f32) + obase
        out_ref[b] = hl


def _h_last(xs2, node_embT, enc_w, enc_b, wq, bq, wk, wv, bv, wo, bo):
    grid = NPAD // BN
    full = lambda shape: pl.BlockSpec(shape, lambda i: (0,) * len(shape))
    return pl.pallas_call(
        _attn_block,
        grid=(grid,),
        in_specs=[
            pl.BlockSpec((B * T, BN), lambda i: (0, i)),
            pl.BlockSpec((H, BN), lambda i: (0, i)),
            full((1, H)), full((H, 1)), full((H, H)), full((H, 1)),
            full((H, H)), full((H, H)), full((H, 1)), full((H, H)), full((1, H)),
        ],
        out_specs=pl.BlockSpec((B, BN, H), lambda i: (0, i, 0)),
        out_shape=jax.ShapeDtypeStruct((B, NPAD, H), jnp.float32),
    )(xs2, node_embT, enc_w, enc_b.reshape(H, 1), wq, bq.reshape(H, 1),
      wk, wv, bv.reshape(H, 1), wo, bo.reshape(1, H))


def _combine_block(s0, s1, s2, s3, s4, diff_w_ref, dec_w_ref, bias_ref, out_ref):
    f32 = jnp.float32
    dg = lax.dot_general
    dec_w = dec_w_ref[...]
    sups = (s0, s1, s2, s3, s4)
    for b in range(B):
        acc = bias_ref[...]
        z = jnp.zeros((HORIZON, s0.shape[1]), f32) + acc
        for i in range(5):
            Ci = dg(diff_w_ref[i * H:(i + 1) * H, :], dec_w,
                    (((1,), (0,)), ((), ())), preferred_element_type=f32)
            z = z + dg(Ci, sups[i][b], (((0,), (1,)), ((), ())),
                       preferred_element_type=f32)
        out_ref[b] = z


def _combine(sups, diff_w, diff_b, dec_w, dec_b):
    bias = (diff_b @ dec_w + dec_b).reshape(HORIZON, 1)
    grid = NPAD // BN
    full = lambda shape: pl.BlockSpec(shape, lambda i: (0,) * len(shape))
    sup_spec = pl.BlockSpec((B, BN, H), lambda i: (0, i, 0))
    return pl.pallas_call(
        _combine_block,
        grid=(grid,),
        in_specs=[sup_spec] * 5 + [full(((2 * K + 1) * H, H)), full((H, HORIZON)),
                                   full((HORIZON, 1))],
        out_specs=pl.BlockSpec((B, HORIZON, BN), lambda i: (0, 0, i)),
        out_shape=jax.ShapeDtypeStruct((B, HORIZON, NPAD), jnp.float32),
    )(*sups, diff_w, dec_w, bias)


def _diffusion(h_last, src, dst, w):
    deg_out = jax.ops.segment_sum(w, src, num_segments=N)
    deg_in = jax.ops.segment_sum(w, dst, num_segments=N)
    dinv_out = jnp.pad(1.0 / jnp.maximum(deg_out, 1e-8), (0, NPAD - N))
    dinv_in = jnp.pad(1.0 / jnp.maximum(deg_in, 1e-8), (0, NPAD - N))

    def hop(h, ig, isc, dinv):
        hs = h * dinv[None, :, None]
        msg = hs[:, ig, :] * w[None, :, None]
        out = jax.vmap(lambda m: jax.ops.segment_sum(m, isc, num_segments=N))(msg)
        return jnp.pad(out, ((0, 0), (0, NPAD - N), (0, 0)))

    hf1 = hop(h_last, src, dst, dinv_out)
    hf2 = hop(hf1, src, dst, dinv_out)
    hb1 = hop(h_last, dst, src, dinv_in)
    hb2 = hop(hb1, dst, src, dinv_in)
    return hf1, hf2, hb1, hb2


@jax.jit
def kernel(x, edge_index, edge_weight, enc_w, enc_b, node_emb, wq, bq, wk, bk,
           wv, bv, wo, bo, diff_w, diff_b, dec_w, dec_b):
    xs2 = jnp.pad(x[..., 0].reshape(B * T, N), ((0, 0), (0, NPAD - N)))
    node_embT = jnp.pad(node_emb.T, ((0, 0), (0, NPAD - N)))
    h_last = _h_last(xs2, node_embT, enc_w, enc_b, wq, bq, wk, wv, bv, wo, bo)
    src, dst = edge_index[0], edge_index[1]
    hf1, hf2, hb1, hb2 = _diffusion(h_last, src, dst, edge_weight)
    out = _combine((h_last, hf1, hf2, hb1, hb2), diff_w, diff_b, dec_w, dec_b)
    return out[:, :, :N, None]


# speedup vs baseline: 64.9031x; 61.6620x over previous
import functools
import jax
import jax.numpy as jnp
from jax import lax
from jax.experimental import pallas as pl
from jax.experimental.pallas import tpu as pltpu
from jax.experimental.pallas import tpu_sc as plsc

B, T, N, F_IN = 2, 12, 50000, 1
E = 1600000
H = 32
HEADS = 8
HD = H // HEADS
HORIZON = 12
K = 2

BN = 1024
NPAD = ((N + BN - 1) // BN) * BN

NTILE = 16
RPT = NPAD // NTILE
EPT = E // NTILE
SUB = 80
CL = 5 * SUB
NIT = EPT // CL
PCH = 112
NPC = RPT // PCH

@functools.cache
def _sc_mesh():
    return plsc.VectorSubcoreMesh(core_axis_name="c", subcore_axis_name="s",
                                  num_cores=2, num_subcores=NTILE)


def _scale_rows(rows, vals_ref, row0, voff, nrows):
    for g in range(nrows // 16):
        v16 = vals_ref[pl.ds(voff + g * 16, 16)]
        for e in range(16):
            r = row0 + g * 16 + e
            ws = v16[e]
            for hh in range(H // 16):
                rows[r, pl.ds(hh * 16, 16)] = rows[r, pl.ds(hh * 16, 16)] * ws


def _attn_block(x_ref, nembT_ref, enc_w_ref, enc_b_ref, wq_ref, bq_ref,
                wk_ref, wv_ref, bv_ref, wo_ref, bo_ref, out_ref):
    f32 = jnp.float32
    dg = lax.dot_general
    embT = nembT_ref[...] + enc_b_ref[...]
    wq = wq_ref[...]
    wv = wv_ref[...]
    wo = wo_ref[...]
    enc_w = enc_w_ref[...]
    qvecC = dg(wq, enc_w, (((0,), (1,)), ((), ())), preferred_element_type=f32)
    kvecC = dg(wk_ref[...], enc_w, (((0,), (1,)), ((), ())), preferred_element_type=f32)
    vvec_row = dg(enc_w, wv, (((1,), (0,)), ((), ())), preferred_element_type=f32)
    qbaseT = dg(wq, embT, (((0,), (0,)), ((), ())), preferred_element_type=f32) + bq_ref[...]
    vbaseT = dg(wv, embT, (((0,), (0,)), ((), ())), preferred_element_type=f32) + bv_ref[...]
    hid = lax.broadcasted_iota(jnp.int32, (HEADS, H), 0)
    cid = lax.broadcasted_iota(jnp.int32, (HEADS, H), 1)
    Ssel = jnp.where(cid // HD == hid, 1.0, 0.0).astype(f32)
    alpha8 = dg(Ssel, qvecC * kvecC, (((1,), (0,)), ((), ())), preferred_element_type=f32)
    betaT = dg(Ssel, qbaseT * kvecC, (((1,), (0,)), ((), ())), preferred_element_type=f32)
    wov = dg(Ssel * vvec_row, wo, (((1,), (0,)), ((), ())), preferred_element_type=f32)
    obase = dg(vbaseT, wo, (((0,), (0,)), ((), ())), preferred_element_type=f32) + bo_ref[...]
    for b in range(B):
        xbt = x_ref[b * T:(b + 1) * T, :]
        xlast = x_ref[b * T + T - 1:b * T + T, :]
        aT = alpha8 * xlast + betaT
        srows = []
        for h in range(HEADS):
            lg = aT[h:h + 1, :] * xbt * 0.5
            m = jnp.max(lg, axis=0, keepdims=True)
            e = jnp.exp(lg - m)
            den = jnp.sum(e, axis=0, keepdims=True)
            srows.append(jnp.sum(e * xbt, axis=0, keepdims=True) / den)
        s8 = jnp.concatenate(srows, axis=0)
        hl = dg(s8, wov, (((0,), (0,)), ((), ())), preferred_element_type=f32) + obase
        out_ref[b] = hl


def _h_last(xs2, node_embT, enc_w, enc_b, wq, bq, wk, wv, bv, wo, bo):
    grid = NPAD // BN
    full = lambda shape: pl.BlockSpec(shape, lambda i: (0,) * len(shape))
    return pl.pallas_call(
        _attn_block,
        grid=(grid,),
        in_specs=[
            pl.BlockSpec((B * T, BN), lambda i: (0, i)),
            pl.BlockSpec((H, BN), lambda i: (0, i)),
            full((1, H)), full((H, 1)), full((H, H)), full((H, 1)),
            full((H, H)), full((H, H)), full((H, 1)), full((H, H)), full((1, H)),
        ],
        out_specs=pl.BlockSpec((B, BN, H), lambda i: (0, i, 0)),
        out_shape=jax.ShapeDtypeStruct((B, NPAD, H), jnp.float32),
    )(xs2, node_embT, enc_w, enc_b.reshape(H, 1), wq, bq.reshape(H, 1),
      wk, wv, bv.reshape(H, 1), wo, bo.reshape(1, H))


def _combine_block(s0, s1, s2, s3, s4, diff_w_ref, dec_w_ref, bias_ref, out_ref):
    f32 = jnp.float32
    dg = lax.dot_general
    dec_w = dec_w_ref[...]
    sups = (s0, s1, s2, s3, s4)
    for b in range(B):
        acc = bias_ref[...]
        z = jnp.zeros((HORIZON, s0.shape[1]), f32) + acc
        for i in range(5):
            Ci = dg(diff_w_ref[i * H:(i + 1) * H, :], dec_w,
                    (((1,), (0,)), ((), ())), preferred_element_type=f32)
            z = z + dg(Ci, sups[i][b], (((0,), (1,)), ((), ())),
                       preferred_element_type=f32)
        out_ref[b] = z


def _combine(sups, diff_w, diff_b, dec_w, dec_b):
    bias = (diff_b @ dec_w + dec_b).reshape(HORIZON, 1)
    grid = NPAD // BN
    full = lambda shape: pl.BlockSpec(shape, lambda i: (0,) * len(shape))
    sup_spec = pl.BlockSpec((B, BN, H), lambda i: (0, i, 0))
    return pl.pallas_call(
        _combine_block,
        grid=(grid,),
        in_specs=[sup_spec] * 5 + [full(((2 * K + 1) * H, H)), full((H, HORIZON)),
                                   full((HORIZON, 1))],
        out_specs=pl.BlockSpec((B, HORIZON, BN), lambda i: (0, 0, i)),
        out_shape=jax.ShapeDtypeStruct((B, HORIZON, NPAD), jnp.float32),
    )(*sups, diff_w, dec_w, bias)


def _deg_body(src_hbm, dst_hbm, w_hbm, dinv_out_hbm, dinv_in_hbm,
              acc, idxb, wb, vb):
    c = lax.axis_index("c")
    s = lax.axis_index("s")
    base = s * RPT
    for i in range(RPT // 16):
        vb[pl.ds(i * 16, 16)] = jnp.zeros((16,), jnp.float32)
    pltpu.sync_copy(vb, acc.at[pl.ds(base, RPT)])
    plsc.subcore_barrier()

    ebase = s * EPT

    @pl.loop(0, NIT)
    def _edges(it):
        off = ebase + it * CL
        for j in range(5):
            @pl.when(c == 0)
            def _():
                pltpu.sync_copy(src_hbm.at[pl.ds(off + j * SUB, SUB)], idxb.at[j])

            @pl.when(c == 1)
            def _():
                pltpu.sync_copy(dst_hbm.at[pl.ds(off + j * SUB, SUB)], idxb.at[j])

            pltpu.sync_copy(w_hbm.at[pl.ds(off + j * SUB, SUB)], wb.at[j])
            pltpu.sync_copy(wb.at[j], acc.at[idxb.at[j]], add=True)

    plsc.subcore_barrier()
    pltpu.sync_copy(acc.at[pl.ds(base, RPT)], vb)
    for i in range(RPT // 16):
        v = vb[pl.ds(i * 16, 16)]
        vb[pl.ds(i * 16, 16)] = 1.0 / jnp.maximum(v, 1e-8)

    @pl.when(c == 0)
    def _():
        pltpu.sync_copy(vb, dinv_out_hbm.at[pl.ds(base, RPT)])

    @pl.when(c == 1)
    def _():
        pltpu.sync_copy(vb, dinv_in_hbm.at[pl.ds(base, RPT)])


@functools.cache
def _deg_call():
    return pl.kernel(
        _deg_body,
        out_type=(jax.ShapeDtypeStruct((NPAD,), jnp.float32),
                  jax.ShapeDtypeStruct((NPAD,), jnp.float32)),
        mesh=_sc_mesh(),
        compiler_params=pltpu.CompilerParams(use_tc_tiling_on_sc=False),
        scratch_types=[
            pltpu.VMEM_SHARED((NPAD,), jnp.float32),
            pltpu.VMEM((5, SUB), jnp.int32),
            pltpu.VMEM((5, SUB), jnp.float32),
            pltpu.VMEM((RPT,), jnp.float32),
        ],
    )


def _hop_body(table_hbm, dinv_hbm, idxg_hbm, idxs_hbm, w_hbm,
              out_hbm, tscr_hbm, acc, idxgb, idxsb, wb, rows, dvb, tb):
    c = lax.axis_index("c")
    s = lax.axis_index("s")
    cN = c * NPAD
    base = s * RPT

    @pl.loop(0, NPC)
    def _pre(pc):
        roff = base + pc * PCH
        pltpu.sync_copy(table_hbm.at[pl.ds(cN + roff, PCH)], tb)
        pltpu.sync_copy(dinv_hbm.at[pl.ds(roff, PCH)], dvb)
        _scale_rows(tb, dvb, 0, 0, PCH)
        pltpu.sync_copy(tb, tscr_hbm.at[pl.ds(cN + roff, PCH)])

    for r in range(PCH):
        for hh in range(2):
            tb[r, pl.ds(hh * 16, 16)] = jnp.zeros((16,), jnp.float32)

    @pl.loop(0, NPC)
    def _zero(pc):
        pltpu.sync_copy(tb, acc.at[pl.ds(base + pc * PCH, PCH)])

    plsc.subcore_barrier()

    ebase = s * EPT

    @pl.loop(0, NIT)
    def _edges(it):
        off = ebase + it * CL
        pltpu.sync_copy(w_hbm.at[pl.ds(off, CL)], wb)
        for j in range(5):
            pltpu.sync_copy(idxs_hbm.at[pl.ds(off + j * SUB, SUB)], idxsb.at[j])
            pltpu.sync_copy(idxg_hbm.at[pl.ds(off + j * SUB, SUB)], idxgb.at[j])
        for j in range(5):
            for g in range(SUB // 16):
                iv = idxgb[j, pl.ds(g * 16, 16)]
                idxgb[j, pl.ds(g * 16, 16)] = iv + cN
        for j in range(5):
            pltpu.sync_copy(tscr_hbm.at[idxgb.at[j]], rows)
            _scale_rows(rows, wb, 0, j * SUB, SUB)
            pltpu.sync_copy(rows, acc.at[idxsb.at[j]], add=True)

    plsc.subcore_barrier()

    @pl.loop(0, NPC)
    def _wb(pc):
        roff = base + pc * PCH
        pltpu.sync_copy(acc.at[pl.ds(roff, PCH)], tb)
        pltpu.sync_copy(tb, out_hbm.at[pl.ds(cN + roff, PCH)])


@functools.cache
def _hop_call():
    return pl.kernel(
        _hop_body,
        out_type=(jax.ShapeDtypeStruct((2 * NPAD, H), jnp.float32),
                  jax.ShapeDtypeStruct((2 * NPAD, H), jnp.float32)),
        mesh=_sc_mesh(),
        compiler_params=pltpu.CompilerParams(use_tc_tiling_on_sc=False),
        scratch_types=[
            pltpu.VMEM_SHARED((NPAD, H), jnp.float32),
            pltpu.VMEM((5, SUB), jnp.int32),
            pltpu.VMEM((5, SUB), jnp.int32),
            pltpu.VMEM((CL,), jnp.float32),
            pltpu.VMEM((SUB, H), jnp.float32),
            pltpu.VMEM((PCH,), jnp.float32),
            pltpu.VMEM((PCH, H), jnp.float32),
        ],
    )


def _diffusion(h_last, src, dst, w):
    dinv_out, dinv_in = _deg_call()(src, dst, w)
    t0 = h_last.reshape(2 * NPAD, H)
    hop = _hop_call()
    hf1, _ = hop(t0, dinv_out, src, dst, w)
    hf2, _ = hop(hf1, dinv_out, src, dst, w)
    hb1, _ = hop(t0, dinv_in, dst, src, w)
    hb2, _ = hop(hb1, dinv_in, dst, src, w)
    return hf1, hf2, hb1, hb2


@jax.jit
def kernel(x, edge_index, edge_weight, enc_w, enc_b, node_emb, wq, bq, wk, bk,
           wv, bv, wo, bo, diff_w, diff_b, dec_w, dec_b):
    xs2 = jnp.pad(x[..., 0].reshape(B * T, N), ((0, 0), (0, NPAD - N)))
    node_embT = jnp.pad(node_emb.T, ((0, 0), (0, NPAD - N)))
    h_last = _h_last(xs2, node_embT, enc_w, enc_b, wq, bq, wk, wv, bv, wo, bo)
    src, dst = edge_index[0], edge_index[1]
    hf1, hf2, hb1, hb2 = _diffusion(h_last, src, dst, edge_weight)
    sups = tuple(a.reshape(B, NPAD, H) for a in
                 (h_last.reshape(2 * NPAD, H), hf1, hf2, hb1, hb2))
    out = _combine(sups, diff_w, diff_b, dec_w, dec_b)
    return out[:, :, :N, None]


# speedup vs baseline: 141.5738x; 2.1813x over previous
import functools
import jax
import jax.numpy as jnp
from jax import lax
from jax.experimental import pallas as pl
from jax.experimental.pallas import tpu as pltpu
from jax.experimental.pallas import tpu_sc as plsc

B, T, N, F_IN = 2, 12, 50000, 1
E = 1600000
H = 32
HEADS = 8
HD = H // HEADS
HORIZON = 12
K = 2

BN = 1024
NPAD = ((N + BN - 1) // BN) * BN

NTILE = 16
RPT = NPAD // NTILE
EPT = E // NTILE
SUB = 80
CL = 5 * SUB
NIT = EPT // CL
PCH = 112
NPC = RPT // PCH

@functools.cache
def _sc_mesh():
    return plsc.VectorSubcoreMesh(core_axis_name="c", subcore_axis_name="s",
                                  num_cores=2, num_subcores=NTILE)


def _scale_rows(rows, vals_ref, row0, voff, nrows):
    for g in range(nrows // 16):
        v16 = vals_ref[pl.ds(voff + g * 16, 16)]
        for e in range(16):
            r = row0 + g * 16 + e
            ws = v16[e]
            for hh in range(H // 16):
                rows[r, pl.ds(hh * 16, 16)] = rows[r, pl.ds(hh * 16, 16)] * ws


def _attn_block(x_ref, nembT_ref, enc_w_ref, enc_b_ref, wq_ref, bq_ref,
                wk_ref, wv_ref, bv_ref, wo_ref, bo_ref, out_ref):
    f32 = jnp.float32
    dg = lax.dot_general
    embT = nembT_ref[...] + enc_b_ref[...]
    wq = wq_ref[...]
    wv = wv_ref[...]
    wo = wo_ref[...]
    enc_w = enc_w_ref[...]
    qvecC = dg(wq, enc_w, (((0,), (1,)), ((), ())), preferred_element_type=f32)
    kvecC = dg(wk_ref[...], enc_w, (((0,), (1,)), ((), ())), preferred_element_type=f32)
    vvec_row = dg(enc_w, wv, (((1,), (0,)), ((), ())), preferred_element_type=f32)
    qbaseT = dg(wq, embT, (((0,), (0,)), ((), ())), preferred_element_type=f32) + bq_ref[...]
    vbaseT = dg(wv, embT, (((0,), (0,)), ((), ())), preferred_element_type=f32) + bv_ref[...]
    hid = lax.broadcasted_iota(jnp.int32, (HEADS, H), 0)
    cid = lax.broadcasted_iota(jnp.int32, (HEADS, H), 1)
    Ssel = jnp.where(cid // HD == hid, 1.0, 0.0).astype(f32)
    alpha8 = dg(Ssel, qvecC * kvecC, (((1,), (0,)), ((), ())), preferred_element_type=f32)
    betaT = dg(Ssel, qbaseT * kvecC, (((1,), (0,)), ((), ())), preferred_element_type=f32)
    wov = dg(Ssel * vvec_row, wo, (((1,), (0,)), ((), ())), preferred_element_type=f32)
    obase = dg(vbaseT, wo, (((0,), (0,)), ((), ())), preferred_element_type=f32) + bo_ref[...]
    for b in range(B):
        xbt = x_ref[b * T:(b + 1) * T, :]
        xlast = x_ref[b * T + T - 1:b * T + T, :]
        aT = alpha8 * xlast + betaT
        srows = []
        for h in range(HEADS):
            lg = aT[h:h + 1, :] * xbt * 0.5
            m = jnp.max(lg, axis=0, keepdims=True)
            e = jnp.exp(lg - m)
            den = jnp.sum(e, axis=0, keepdims=True)
            srows.append(jnp.sum(e * xbt, axis=0, keepdims=True) / den)
        s8 = jnp.concatenate(srows, axis=0)
        hl = dg(s8, wov, (((0,), (0,)), ((), ())), preferred_element_type=f32) + obase
        out_ref[b] = hl


def _h_last(xs2, node_embT, enc_w, enc_b, wq, bq, wk, wv, bv, wo, bo):
    grid = NPAD // BN
    full = lambda shape: pl.BlockSpec(shape, lambda i: (0,) * len(shape))
    return pl.pallas_call(
        _attn_block,
        grid=(grid,),
        in_specs=[
            pl.BlockSpec((B * T, BN), lambda i: (0, i)),
            pl.BlockSpec((H, BN), lambda i: (0, i)),
            full((1, H)), full((H, 1)), full((H, H)), full((H, 1)),
            full((H, H)), full((H, H)), full((H, 1)), full((H, H)), full((1, H)),
        ],
        out_specs=pl.BlockSpec((B, BN, H), lambda i: (0, i, 0)),
        out_shape=jax.ShapeDtypeStruct((B, NPAD, H), jnp.float32),
    )(xs2, node_embT, enc_w, enc_b.reshape(H, 1), wq, bq.reshape(H, 1),
      wk, wv, bv.reshape(H, 1), wo, bo.reshape(1, H))


def _combine_block(s0, s1, s2, s3, s4, diff_w_ref, dec_w_ref, bias_ref, out_ref):
    f32 = jnp.float32
    dg = lax.dot_general
    dec_w = dec_w_ref[...]
    sups = (s0, s1, s2, s3, s4)
    for b in range(B):
        acc = bias_ref[...]
        z = jnp.zeros((HORIZON, s0.shape[1]), f32) + acc
        for i in range(5):
            Ci = dg(diff_w_ref[i * H:(i + 1) * H, :], dec_w,
                    (((1,), (0,)), ((), ())), preferred_element_type=f32)
            z = z + dg(Ci, sups[i][b], (((0,), (1,)), ((), ())),
                       preferred_element_type=f32)
        out_ref[b] = z


def _combine(sups, diff_w, diff_b, dec_w, dec_b):
    bias = (diff_b @ dec_w + dec_b).reshape(HORIZON, 1)
    grid = NPAD // BN
    full = lambda shape: pl.BlockSpec(shape, lambda i: (0,) * len(shape))
    sup_spec = pl.BlockSpec((B, BN, H), lambda i: (0, i, 0))
    return pl.pallas_call(
        _combine_block,
        grid=(grid,),
        in_specs=[sup_spec] * 5 + [full(((2 * K + 1) * H, H)), full((H, HORIZON)),
                                   full((HORIZON, 1))],
        out_specs=pl.BlockSpec((B, HORIZON, BN), lambda i: (0, 0, i)),
        out_shape=jax.ShapeDtypeStruct((B, HORIZON, NPAD), jnp.float32),
    )(*sups, diff_w, dec_w, bias)


def _deg_body(src_hbm, dst_hbm, w_hbm, dinv_out_hbm, dinv_in_hbm,
              acc_s, acc_d, idxsb, idxdb, wb, vb, lsem, ssem):
    c = lax.axis_index("c")
    s = lax.axis_index("s")
    base = s * RPT
    for i in range(RPT // 16):
        vb[pl.ds(i * 16, 16)] = jnp.zeros((16,), jnp.float32)
    pltpu.sync_copy(vb, acc_s.at[pl.ds(base, RPT)])
    pltpu.sync_copy(vb, acc_d.at[pl.ds(base, RPT)])
    plsc.subcore_barrier()

    ebase = s * EPT

    @pl.loop(0, NIT)
    def _edges(it):
        off = ebase + it * CL
        ld = [pltpu.async_copy(w_hbm.at[pl.ds(off, CL)], wb, lsem)]
        for j in range(5):
            sl = pl.ds(off + j * SUB, SUB)
            ld.append(pltpu.async_copy(src_hbm.at[sl], idxsb.at[j], lsem))
            ld.append(pltpu.async_copy(dst_hbm.at[sl], idxdb.at[j], lsem))
        for h in ld:
            h.wait()
        sh = []
        for j in range(5):
            wsl = wb.at[pl.ds(j * SUB, SUB)]
            sh.append(pltpu.async_copy(wsl, acc_s.at[idxsb.at[j]], ssem, add=True))
            sh.append(pltpu.async_copy(wsl, acc_d.at[idxdb.at[j]], ssem, add=True))
        for h in sh:
            h.wait()

    plsc.subcore_barrier()

    @pl.when(c == 0)
    def _():
        pltpu.sync_copy(acc_s.at[pl.ds(base, RPT)], vb)

    @pl.when(c == 1)
    def _():
        pltpu.sync_copy(acc_d.at[pl.ds(base, RPT)], vb)

    for i in range(RPT // 16):
        v = vb[pl.ds(i * 16, 16)]
        vb[pl.ds(i * 16, 16)] = 1.0 / jnp.maximum(v, 1e-8)

    @pl.when(c == 0)
    def _():
        pltpu.sync_copy(vb, dinv_out_hbm.at[pl.ds(base, RPT)])

    @pl.when(c == 1)
    def _():
        pltpu.sync_copy(vb, dinv_in_hbm.at[pl.ds(base, RPT)])


@functools.cache
def _deg_call():
    return pl.kernel(
        _deg_body,
        out_type=(jax.ShapeDtypeStruct((NPAD,), jnp.float32),
                  jax.ShapeDtypeStruct((NPAD,), jnp.float32)),
        mesh=_sc_mesh(),
        compiler_params=pltpu.CompilerParams(use_tc_tiling_on_sc=False),
        scratch_types=[
            pltpu.VMEM_SHARED((NPAD,), jnp.float32),
            pltpu.VMEM_SHARED((NPAD,), jnp.float32),
            pltpu.VMEM((5, SUB), jnp.int32),
            pltpu.VMEM((5, SUB), jnp.int32),
            pltpu.VMEM((CL,), jnp.float32),
            pltpu.VMEM((RPT,), jnp.float32),
            pltpu.SemaphoreType.DMA,
            pltpu.SemaphoreType.DMA,
        ],
    )


def _hop_body(table_hbm, dinv_hbm, idxg_hbm, idxs_hbm, w_hbm,
              out_hbm, tscr_hbm, acc, idxgb, idxsb, wb, rows0, rows1, dvb, tb,
              lsem, gsem, ssem):
    c = lax.axis_index("c")
    s = lax.axis_index("s")
    cN = c * NPAD
    base = s * RPT

    @pl.loop(0, NPC)
    def _pre(pc):
        roff = base + pc * PCH
        pltpu.sync_copy(table_hbm.at[pl.ds(cN + roff, PCH)], tb)
        pltpu.sync_copy(dinv_hbm.at[pl.ds(roff, PCH)], dvb)
        _scale_rows(tb, dvb, 0, 0, PCH)
        pltpu.sync_copy(tb, tscr_hbm.at[pl.ds(cN + roff, PCH)])

    for r in range(PCH):
        for hh in range(2):
            tb[r, pl.ds(hh * 16, 16)] = jnp.zeros((16,), jnp.float32)

    @pl.loop(0, NPC)
    def _zero(pc):
        pltpu.sync_copy(tb, acc.at[pl.ds(base + pc * PCH, PCH)])

    plsc.subcore_barrier()

    ebase = s * EPT

    @pl.loop(0, NIT)
    def _edges(it):
        off = ebase + it * CL
        ld = [pltpu.async_copy(w_hbm.at[pl.ds(off, CL)], wb, lsem)]
        for j in range(5):
            sl = pl.ds(off + j * SUB, SUB)
            ld.append(pltpu.async_copy(idxs_hbm.at[sl], idxsb.at[j], lsem))
            ld.append(pltpu.async_copy(idxg_hbm.at[sl], idxgb.at[j], lsem))
        for h in ld:
            h.wait()
        for j in range(5):
            for g in range(SUB // 16):
                iv = idxgb[j, pl.ds(g * 16, 16)]
                idxgb[j, pl.ds(g * 16, 16)] = iv + cN
        rbufs = (rows0, rows1)
        gh = [None] * 5
        sh = [None] * 5
        gh[0] = pltpu.async_copy(tscr_hbm.at[idxgb.at[0]], rows0, gsem)
        for j in range(5):
            rb = rbufs[j & 1]
            gh[j].wait()
            if j < 4:
                if j >= 1:
                    sh[j - 1].wait()
                gh[j + 1] = pltpu.async_copy(
                    tscr_hbm.at[idxgb.at[j + 1]], rbufs[(j + 1) & 1], gsem)
            _scale_rows(rb, wb, 0, j * SUB, SUB)
            sh[j] = pltpu.async_copy(rb, acc.at[idxsb.at[j]], ssem, add=True)
        sh[3].wait()
        sh[4].wait()

    plsc.subcore_barrier()

    @pl.loop(0, NPC)
    def _wb(pc):
        roff = base + pc * PCH
        pltpu.sync_copy(acc.at[pl.ds(roff, PCH)], tb)
        pltpu.sync_copy(tb, out_hbm.at[pl.ds(cN + roff, PCH)])


@functools.cache
def _hop_call():
    return pl.kernel(
        _hop_body,
        out_type=(jax.ShapeDtypeStruct((2 * NPAD, H), jnp.float32),
                  jax.ShapeDtypeStruct((2 * NPAD, H), jnp.float32)),
        mesh=_sc_mesh(),
        compiler_params=pltpu.CompilerParams(use_tc_tiling_on_sc=False),
        scratch_types=[
            pltpu.VMEM_SHARED((NPAD, H), jnp.float32),
            pltpu.VMEM((5, SUB), jnp.int32),
            pltpu.VMEM((5, SUB), jnp.int32),
            pltpu.VMEM((CL,), jnp.float32),
            pltpu.VMEM((SUB, H), jnp.float32),
            pltpu.VMEM((SUB, H), jnp.float32),
            pltpu.VMEM((PCH,), jnp.float32),
            pltpu.VMEM((PCH, H), jnp.float32),
            pltpu.SemaphoreType.DMA,
            pltpu.SemaphoreType.DMA,
            pltpu.SemaphoreType.DMA,
        ],
    )


def _diffusion(h_last, src, dst, w):
    dinv_out, dinv_in = _deg_call()(src, dst, w)
    t0 = h_last.reshape(2 * NPAD, H)
    hop = _hop_call()
    hf1, _ = hop(t0, dinv_out, src, dst, w)
    hf2, _ = hop(hf1, dinv_out, src, dst, w)
    hb1, _ = hop(t0, dinv_in, dst, src, w)
    hb2, _ = hop(hb1, dinv_in, dst, src, w)
    return hf1, hf2, hb1, hb2


@jax.jit
def kernel(x, edge_index, edge_weight, enc_w, enc_b, node_emb, wq, bq, wk, bk,
           wv, bv, wo, bo, diff_w, diff_b, dec_w, dec_b):
    xs2 = jnp.pad(x[..., 0].reshape(B * T, N), ((0, 0), (0, NPAD - N)))
    node_embT = jnp.pad(node_emb.T, ((0, 0), (0, NPAD - N)))
    h_last = _h_last(xs2, node_embT, enc_w, enc_b, wq, bq, wk, wv, bv, wo, bo)
    src, dst = edge_index[0], edge_index[1]
    hf1, hf2, hb1, hb2 = _diffusion(h_last, src, dst, edge_weight)
    sups = tuple(a.reshape(B, NPAD, H) for a in
                 (h_last.reshape(2 * NPAD, H), hf1, hf2, hb1, hb2))
    out = _combine(sups, diff_w, diff_b, dec_w, dec_b)
    return out[:, :, :N, None]


# speedup vs baseline: 247.7766x; 1.7502x over previous
import functools
import jax
import jax.numpy as jnp
from jax import lax
from jax.experimental import pallas as pl
from jax.experimental.pallas import tpu as pltpu
from jax.experimental.pallas import tpu_sc as plsc

B, T, N, F_IN = 2, 12, 50000, 1
E = 1600000
H = 32
HEADS = 8
HD = H // HEADS
HORIZON = 12
K = 2

BN = 1024
NPAD = ((N + BN - 1) // BN) * BN

NTILE = 16
RPT = NPAD // NTILE
EPT = E // NTILE
SUB = 80
CL = 5 * SUB
NIT = EPT // CL
PCH = 112
NPC = RPT // PCH

@functools.cache
def _sc_mesh():
    return plsc.VectorSubcoreMesh(core_axis_name="c", subcore_axis_name="s",
                                  num_cores=2, num_subcores=NTILE)


def _scale_rows(rows, vals_ref, row0, voff, nrows):
    for g in range(nrows // 16):
        v16 = vals_ref[pl.ds(voff + g * 16, 16)]
        for e in range(16):
            r = row0 + g * 16 + e
            ws = v16[e]
            for hh in range(H // 16):
                rows[r, pl.ds(hh * 16, 16)] = rows[r, pl.ds(hh * 16, 16)] * ws


def _attn_block(x_ref, nembT_ref, enc_w_ref, enc_b_ref, wq_ref, bq_ref,
                wk_ref, wv_ref, bv_ref, wo_ref, bo_ref, out_ref):
    f32 = jnp.float32
    dg = lax.dot_general
    embT = nembT_ref[...] + enc_b_ref[...]
    wq = wq_ref[...]
    wv = wv_ref[...]
    wo = wo_ref[...]
    enc_w = enc_w_ref[...]
    qvecC = dg(wq, enc_w, (((0,), (1,)), ((), ())), preferred_element_type=f32)
    kvecC = dg(wk_ref[...], enc_w, (((0,), (1,)), ((), ())), preferred_element_type=f32)
    vvec_row = dg(enc_w, wv, (((1,), (0,)), ((), ())), preferred_element_type=f32)
    qbaseT = dg(wq, embT, (((0,), (0,)), ((), ())), preferred_element_type=f32) + bq_ref[...]
    vbaseT = dg(wv, embT, (((0,), (0,)), ((), ())), preferred_element_type=f32) + bv_ref[...]
    hid = lax.broadcasted_iota(jnp.int32, (HEADS, H), 0)
    cid = lax.broadcasted_iota(jnp.int32, (HEADS, H), 1)
    Ssel = jnp.where(cid // HD == hid, 1.0, 0.0).astype(f32)
    alpha8 = dg(Ssel, qvecC * kvecC, (((1,), (0,)), ((), ())), preferred_element_type=f32)
    betaT = dg(Ssel, qbaseT * kvecC, (((1,), (0,)), ((), ())), preferred_element_type=f32)
    wov = dg(Ssel * vvec_row, wo, (((1,), (0,)), ((), ())), preferred_element_type=f32)
    obase = dg(vbaseT, wo, (((0,), (0,)), ((), ())), preferred_element_type=f32) + bo_ref[...]
    for b in range(B):
        xbt = x_ref[b * T:(b + 1) * T, :]
        xlast = x_ref[b * T + T - 1:b * T + T, :]
        aT = alpha8 * xlast + betaT
        srows = []
        for h in range(HEADS):
            lg = aT[h:h + 1, :] * xbt * 0.5
            m = jnp.max(lg, axis=0, keepdims=True)
            e = jnp.exp(lg - m)
            den = jnp.sum(e, axis=0, keepdims=True)
            srows.append(jnp.sum(e * xbt, axis=0, keepdims=True) / den)
        s8 = jnp.concatenate(srows, axis=0)
        hl = dg(s8, wov, (((0,), (0,)), ((), ())), preferred_element_type=f32) + obase
        out_ref[b] = hl


def _h_last(xs2, node_embT, enc_w, enc_b, wq, bq, wk, wv, bv, wo, bo):
    grid = NPAD // BN
    full = lambda shape: pl.BlockSpec(shape, lambda i: (0,) * len(shape))
    return pl.pallas_call(
        _attn_block,
        grid=(grid,),
        in_specs=[
            pl.BlockSpec((B * T, BN), lambda i: (0, i)),
            pl.BlockSpec((H, BN), lambda i: (0, i)),
            full((1, H)), full((H, 1)), full((H, H)), full((H, 1)),
            full((H, H)), full((H, H)), full((H, 1)), full((H, H)), full((1, H)),
        ],
        out_specs=pl.BlockSpec((B, BN, H), lambda i: (0, i, 0)),
        out_shape=jax.ShapeDtypeStruct((B, NPAD, H), jnp.float32),
    )(xs2, node_embT, enc_w, enc_b.reshape(H, 1), wq, bq.reshape(H, 1),
      wk, wv, bv.reshape(H, 1), wo, bo.reshape(1, H))


def _combine_block(s0, s1, s2, s3, s4, diff_w_ref, dec_w_ref, bias_ref, out_ref):
    f32 = jnp.float32
    dg = lax.dot_general
    dec_w = dec_w_ref[...]
    sups = (s0, s1, s2, s3, s4)
    for b in range(B):
        acc = bias_ref[...]
        z = jnp.zeros((HORIZON, s0.shape[1]), f32) + acc
        for i in range(5):
            Ci = dg(diff_w_ref[i * H:(i + 1) * H, :], dec_w,
                    (((1,), (0,)), ((), ())), preferred_element_type=f32)
            z = z + dg(Ci, sups[i][b], (((0,), (1,)), ((), ())),
                       preferred_element_type=f32)
        out_ref[b] = z


def _combine(sups, diff_w, diff_b, dec_w, dec_b):
    bias = (diff_b @ dec_w + dec_b).reshape(HORIZON, 1)
    grid = NPAD // BN
    full = lambda shape: pl.BlockSpec(shape, lambda i: (0,) * len(shape))
    sup_spec = pl.BlockSpec((B, BN, H), lambda i: (0, i, 0))
    return pl.pallas_call(
        _combine_block,
        grid=(grid,),
        in_specs=[sup_spec] * 5 + [full(((2 * K + 1) * H, H)), full((H, HORIZON)),
                                   full((HORIZON, 1))],
        out_specs=pl.BlockSpec((B, HORIZON, BN), lambda i: (0, 0, i)),
        out_shape=jax.ShapeDtypeStruct((B, HORIZON, NPAD), jnp.float32),
    )(*sups, diff_w, dec_w, bias)


def _deg_body(src_hbm, dst_hbm, w_hbm, dinv_out_hbm, dinv_in_hbm,
              acc_s, acc_d, idxsb, idxdb, wb, vb, lsem, ssem):
    c = lax.axis_index("c")
    s = lax.axis_index("s")
    base = s * RPT
    for i in range(RPT // 16):
        vb[pl.ds(i * 16, 16)] = jnp.zeros((16,), jnp.float32)
    pltpu.sync_copy(vb, acc_s.at[pl.ds(base, RPT)])
    pltpu.sync_copy(vb, acc_d.at[pl.ds(base, RPT)])
    plsc.subcore_barrier()

    ebase = s * EPT

    @pl.loop(0, NIT)
    def _edges(it):
        off = ebase + it * CL
        ld = [pltpu.async_copy(w_hbm.at[pl.ds(off, CL)], wb, lsem)]
        for j in range(5):
            sl = pl.ds(off + j * SUB, SUB)
            ld.append(pltpu.async_copy(src_hbm.at[sl], idxsb.at[j], lsem))
            ld.append(pltpu.async_copy(dst_hbm.at[sl], idxdb.at[j], lsem))
        for h in ld:
            h.wait()
        sh = []
        for j in range(5):
            wsl = wb.at[pl.ds(j * SUB, SUB)]
            sh.append(pltpu.async_copy(wsl, acc_s.at[idxsb.at[j]], ssem, add=True))
            sh.append(pltpu.async_copy(wsl, acc_d.at[idxdb.at[j]], ssem, add=True))
        for h in sh:
            h.wait()

    plsc.subcore_barrier()

    @pl.when(c == 0)
    def _():
        pltpu.sync_copy(acc_s.at[pl.ds(base, RPT)], vb)

    @pl.when(c == 1)
    def _():
        pltpu.sync_copy(acc_d.at[pl.ds(base, RPT)], vb)

    for i in range(RPT // 16):
        v = vb[pl.ds(i * 16, 16)]
        vb[pl.ds(i * 16, 16)] = 1.0 / jnp.maximum(v, 1e-8)

    @pl.when(c == 0)
    def _():
        pltpu.sync_copy(vb, dinv_out_hbm.at[pl.ds(base, RPT)])

    @pl.when(c == 1)
    def _():
        pltpu.sync_copy(vb, dinv_in_hbm.at[pl.ds(base, RPT)])


@functools.cache
def _deg_call():
    return pl.kernel(
        _deg_body,
        out_type=(jax.ShapeDtypeStruct((NPAD,), jnp.float32),
                  jax.ShapeDtypeStruct((NPAD,), jnp.float32)),
        mesh=_sc_mesh(),
        compiler_params=pltpu.CompilerParams(use_tc_tiling_on_sc=False),
        scratch_types=[
            pltpu.VMEM_SHARED((NPAD,), jnp.float32),
            pltpu.VMEM_SHARED((NPAD,), jnp.float32),
            pltpu.VMEM((5, SUB), jnp.int32),
            pltpu.VMEM((5, SUB), jnp.int32),
            pltpu.VMEM((CL,), jnp.float32),
            pltpu.VMEM((RPT,), jnp.float32),
            pltpu.SemaphoreType.DMA,
            pltpu.SemaphoreType.DMA,
        ],
    )


def _hop_body(table_hbm, dinv_hbm, idxg_hbm, idxs_hbm, w_hbm,
              out_hbm, tscr_hbm, acc, idxgb, idxsb, wb,
              rows0, rows1, rows2, rows3, rows4, dvb, tb,
              lsem, gsem, ssem):
    c = lax.axis_index("c")
    s = lax.axis_index("s")
    cN = c * NPAD
    base = s * RPT

    @pl.loop(0, NPC)
    def _pre(pc):
        roff = base + pc * PCH
        pltpu.sync_copy(table_hbm.at[pl.ds(cN + roff, PCH)], tb)
        pltpu.sync_copy(dinv_hbm.at[pl.ds(roff, PCH)], dvb)
        _scale_rows(tb, dvb, 0, 0, PCH)
        pltpu.sync_copy(tb, tscr_hbm.at[pl.ds(cN + roff, PCH)])

    for r in range(PCH):
        for hh in range(2):
            tb[r, pl.ds(hh * 16, 16)] = jnp.zeros((16,), jnp.float32)

    @pl.loop(0, NPC)
    def _zero(pc):
        pltpu.sync_copy(tb, acc.at[pl.ds(base + pc * PCH, PCH)])

    plsc.subcore_barrier()

    ebase = s * EPT

    @pl.loop(0, NIT)
    def _edges(it):
        off = ebase + it * CL
        ld = [pltpu.async_copy(w_hbm.at[pl.ds(off, CL)], wb, lsem)]
        for j in range(5):
            sl = pl.ds(off + j * SUB, SUB)
            ld.append(pltpu.async_copy(idxs_hbm.at[sl], idxsb.at[j], lsem))
            ld.append(pltpu.async_copy(idxg_hbm.at[sl], idxgb.at[j], lsem))
        for h in ld:
            h.wait()
        for j in range(5):
            for g in range(SUB // 16):
                iv = idxgb[j, pl.ds(g * 16, 16)]
                idxgb[j, pl.ds(g * 16, 16)] = iv + cN
        rbufs = (rows0, rows1, rows2, rows3, rows4)
        gh = [pltpu.async_copy(tscr_hbm.at[idxgb.at[j]], rbufs[j], gsem)
              for j in range(5)]
        sh = []
        for j in range(5):
            gh[j].wait()
            _scale_rows(rbufs[j], wb, 0, j * SUB, SUB)
            sh.append(pltpu.async_copy(rbufs[j], acc.at[idxsb.at[j]], ssem,
                                       add=True))
        for h in sh:
            h.wait()

    plsc.subcore_barrier()

    @pl.loop(0, NPC)
    def _wb(pc):
        roff = base + pc * PCH
        pltpu.sync_copy(acc.at[pl.ds(roff, PCH)], tb)
        pltpu.sync_copy(tb, out_hbm.at[pl.ds(cN + roff, PCH)])


@functools.cache
def _hop_call():
    return pl.kernel(
        _hop_body,
        out_type=(jax.ShapeDtypeStruct((2 * NPAD, H), jnp.float32),
                  jax.ShapeDtypeStruct((2 * NPAD, H), jnp.float32)),
        mesh=_sc_mesh(),
        compiler_params=pltpu.CompilerParams(use_tc_tiling_on_sc=False),
        scratch_types=[
            pltpu.VMEM_SHARED((NPAD, H), jnp.float32),
            pltpu.VMEM((5, SUB), jnp.int32),
            pltpu.VMEM((5, SUB), jnp.int32),
            pltpu.VMEM((CL,), jnp.float32),
            pltpu.VMEM((SUB, H), jnp.float32),
            pltpu.VMEM((SUB, H), jnp.float32),
            pltpu.VMEM((SUB, H), jnp.float32),
            pltpu.VMEM((SUB, H), jnp.float32),
            pltpu.VMEM((SUB, H), jnp.float32),
            pltpu.VMEM((PCH,), jnp.float32),
            pltpu.VMEM((PCH, H), jnp.float32),
            pltpu.SemaphoreType.DMA,
            pltpu.SemaphoreType.DMA,
            pltpu.SemaphoreType.DMA,
        ],
    )


def _diffusion(h_last, src, dst, w):
    dinv_out, dinv_in = _deg_call()(src, dst, w)
    t0 = h_last.reshape(2 * NPAD, H)
    hop = _hop_call()
    hf1, _ = hop(t0, dinv_out, src, dst, w)
    hf2, _ = hop(hf1, dinv_out, src, dst, w)
    hb1, _ = hop(t0, dinv_in, dst, src, w)
    hb2, _ = hop(hb1, dinv_in, dst, src, w)
    return hf1, hf2, hb1, hb2


@jax.jit
def kernel(x, edge_index, edge_weight, enc_w, enc_b, node_emb, wq, bq, wk, bk,
           wv, bv, wo, bo, diff_w, diff_b, dec_w, dec_b):
    xs2 = jnp.pad(x[..., 0].reshape(B * T, N), ((0, 0), (0, NPAD - N)))
    node_embT = jnp.pad(node_emb.T, ((0, 0), (0, NPAD - N)))
    h_last = _h_last(xs2, node_embT, enc_w, enc_b, wq, bq, wk, wv, bv, wo, bo)
    src, dst = edge_index[0], edge_index[1]
    hf1, hf2, hb1, hb2 = _diffusion(h_last, src, dst, edge_weight)
    sups = tuple(a.reshape(B, NPAD, H) for a in
                 (h_last.reshape(2 * NPAD, H), hf1, hf2, hb1, hb2))
    out = _combine(sups, diff_w, diff_b, dec_w, dec_b)
    return out[:, :, :N, None]
